# Initial kernel scaffold; baseline (speedup 1.0000x reference)
import jax
import jax.numpy as jnp
from jax.experimental import pallas as pl


def kernel(x, edge_index, edge_attr, batch, mw, frac, salt_mol, salt_x, salt_edge_index, salt_edge_attr, salt_batch, params):
    raise NotImplementedError("write your pallas kernel here")


# speedup vs baseline: 3.9738x; 3.9738x over previous
import functools

import jax
import jax.numpy as jnp
from jax import lax
from jax.experimental import pallas as pl
from jax.experimental.pallas import tpu as pltpu
from jax.experimental.pallas import tpu_sc as plsc

N_NODES = 10000
N_EDGES = 320000
D = 128
NG = 32
SN = 50
SE = 100
SN_PAD = 64
SE_PAD = 128

NW = 32
EPW = N_EDGES // NW
CHUNK = 80
NCHUNK = EPW // CHUNK
RPT = N_NODES // 16


def _edge_aggregate(x, src, dst, ew):
  mesh = plsc.VectorSubcoreMesh(core_axis_name="c", subcore_axis_name="s")

  @functools.partial(
      pl.kernel,
      mesh=mesh,
      out_type=jax.ShapeDtypeStruct((2 * N_NODES, D), jnp.float32),
      scratch_types=[
          pltpu.VMEM((CHUNK,), jnp.int32),
          pltpu.VMEM((CHUNK,), jnp.int32),
          pltpu.VMEM((CHUNK,), jnp.float32),
          pltpu.VMEM((CHUNK, D), jnp.float32),
          pltpu.VMEM((200, D), jnp.float32),
          pltpu.VMEM_SHARED((N_NODES, D), jnp.float32),
          pltpu.SemaphoreType.DMA,
      ],
  )
  def k(x_hbm, src_hbm, dst_hbm, ew_hbm, out_hbm,
        src_v, dst_v, ew_v, rows_v, zero_v, aggr_sh, sem):
    cid = lax.axis_index("c")
    sid = lax.axis_index("s")
    wid = sid * 2 + cid

    zeros = jnp.zeros((16,), jnp.float32)

    def zrow(r, _):
      for j in range(8):
        zero_v[r, pl.ds(j * 16, 16)] = zeros
      return 0
    lax.fori_loop(0, 200, zrow, 0)

    for kk in range(3):
      pltpu.sync_copy(zero_v, aggr_sh.at[pl.ds(kk * 3200 + sid * 200, 200)])

    @pl.when(sid < 2)
    def _():
      pltpu.sync_copy(zero_v, aggr_sh.at[pl.ds(9600 + sid * 200, 200)])
    plsc.subcore_barrier()

    base0 = wid * EPW

    def chunk_body(i, _):
      base = base0 + i * CHUNK
      pltpu.sync_copy(src_hbm.at[pl.ds(base, CHUNK)], src_v)
      pltpu.sync_copy(dst_hbm.at[pl.ds(base, CHUNK)], dst_v)
      pltpu.sync_copy(ew_hbm.at[pl.ds(base, CHUNK)], ew_v)
      pltpu.async_copy(x_hbm.at[src_v], rows_v, sem).wait()

      def scale_group(g, _):
        wv = ew_v[pl.ds(g * 16, 16)]
        for u in range(16):
          e = g * 16 + u
          w = lax.gather(
              wv, jnp.full((16, 1), u, jnp.int32),
              lax.GatherDimensionNumbers(offset_dims=(),
                                         collapsed_slice_dims=(0,),
                                         start_index_map=(0,)),
              slice_sizes=(1,),
              mode=lax.GatherScatterMode.PROMISE_IN_BOUNDS)
          for j in range(8):
            rows_v[e, pl.ds(j * 16, 16)] = rows_v[e, pl.ds(j * 16, 16)] * w
        return 0
      lax.fori_loop(0, CHUNK // 16, scale_group, 0)

      pltpu.sync_copy(rows_v, aggr_sh.at[dst_v], add=True)
      return 0
    lax.fori_loop(0, NCHUNK, chunk_body, 0)

    plsc.subcore_barrier()
    obase = cid * N_NODES
    for kk in range(3):
      pltpu.sync_copy(aggr_sh.at[pl.ds(kk * 3200 + sid * 200, 200)],
                      out_hbm.at[pl.ds(obase + kk * 3200 + sid * 200, 200)])

    @pl.when(sid < 2)
    def _():
      pltpu.sync_copy(aggr_sh.at[pl.ds(9600 + sid * 200, 200)],
                      out_hbm.at[pl.ds(obase + 9600 + sid * 200, 200)])

  return k(x, src, dst, ew)


BR = 2000
GRID = N_NODES // BR


def _conv_body(a0_ref, a1_ref, x_ref, wr_ref, br_ref, wroot_ref, g_ref, b_ref,
               o_ref):
  a = a0_ref[...] + a1_ref[...]
  y = jnp.dot(a, wr_ref[...], preferred_element_type=jnp.float32)
  y = y + jnp.dot(x_ref[...], wroot_ref[...], preferred_element_type=jnp.float32)
  y = y + br_ref[...]
  mu = jnp.mean(y, axis=1, keepdims=True)
  var = jnp.mean((y - mu) ** 2, axis=1, keepdims=True)
  y = (y - mu) * lax.rsqrt(var + 1e-5) * g_ref[...] + b_ref[...]
  o_ref[...] = jnp.maximum(y, 0.0)


def _conv_dense(aggr2, x, wrT, br_, wrootT, g_, b_):
  a0 = aggr2[:N_NODES]
  a1 = aggr2[N_NODES:]
  return pl.pallas_call(
      _conv_body,
      grid=(GRID,),
      in_specs=[
          pl.BlockSpec((BR, D), lambda i: (i, 0)),
          pl.BlockSpec((BR, D), lambda i: (i, 0)),
          pl.BlockSpec((BR, D), lambda i: (i, 0)),
          pl.BlockSpec((D, D), lambda i: (0, 0)),
          pl.BlockSpec((1, D), lambda i: (0, 0)),
          pl.BlockSpec((D, D), lambda i: (0, 0)),
          pl.BlockSpec((1, D), lambda i: (0, 0)),
          pl.BlockSpec((1, D), lambda i: (0, 0)),
      ],
      out_specs=pl.BlockSpec((BR, D), lambda i: (i, 0)),
      out_shape=jax.ShapeDtypeStruct((N_NODES, D), jnp.float32),
  )(a0, a1, x, wrT, br_, wrootT, g_, b_)


def _conv_pool_body(a0_ref, a1_ref, x_ref, wr_ref, br_ref, wroot_ref, g_ref,
                    b_ref, bat_ref, sums_ref, cnt_ref, acc, cacc):
  i = pl.program_id(0)
  a = a0_ref[...] + a1_ref[...]
  y = jnp.dot(a, wr_ref[...], preferred_element_type=jnp.float32)
  y = y + jnp.dot(x_ref[...], wroot_ref[...], preferred_element_type=jnp.float32)
  y = y + br_ref[...]
  mu = jnp.mean(y, axis=1, keepdims=True)
  var = jnp.mean((y - mu) ** 2, axis=1, keepdims=True)
  y = (y - mu) * lax.rsqrt(var + 1e-5) * g_ref[...] + b_ref[...]
  h = jnp.maximum(y, 0.0)

  bat = bat_ref[0, 0, :]
  onehot = (bat[:, None] == lax.broadcasted_iota(jnp.int32, (BR, NG), 1))
  onehot = onehot.astype(jnp.float32)
  psum = lax.dot_general(onehot, h, (((0,), (0,)), ((), ())),
                         preferred_element_type=jnp.float32)
  pcnt = lax.dot_general(onehot, jnp.ones((BR, D), jnp.float32),
                         (((0,), (0,)), ((), ())),
                         preferred_element_type=jnp.float32)

  @pl.when(i == 0)
  def _():
    acc[...] = jnp.zeros_like(acc)
    cacc[...] = jnp.zeros_like(cacc)

  acc[...] += psum
  cacc[...] += pcnt

  @pl.when(i == GRID - 1)
  def _():
    sums_ref[...] = acc[...]
    cnt_ref[...] = cacc[...]


def _conv_pool(aggr2, x, wrT, br_, wrootT, g_, b_, batch3):
  a0 = aggr2[:N_NODES]
  a1 = aggr2[N_NODES:]
  return pl.pallas_call(
      _conv_pool_body,
      grid=(GRID,),
      in_specs=[
          pl.BlockSpec((BR, D), lambda i: (i, 0)),
          pl.BlockSpec((BR, D), lambda i: (i, 0)),
          pl.BlockSpec((BR, D), lambda i: (i, 0)),
          pl.BlockSpec((D, D), lambda i: (0, 0)),
          pl.BlockSpec((1, D), lambda i: (0, 0)),
          pl.BlockSpec((D, D), lambda i: (0, 0)),
          pl.BlockSpec((1, D), lambda i: (0, 0)),
          pl.BlockSpec((1, D), lambda i: (0, 0)),
          pl.BlockSpec((1, 1, BR), lambda i: (i, 0, 0)),
      ],
      out_specs=[
          pl.BlockSpec((NG, D), lambda i: (0, 0)),
          pl.BlockSpec((NG, D), lambda i: (0, 0)),
      ],
      out_shape=[
          jax.ShapeDtypeStruct((NG, D), jnp.float32),
          jax.ShapeDtypeStruct((NG, D), jnp.float32),
      ],
      scratch_shapes=[
          pltpu.VMEM((NG, D), jnp.float32),
          pltpu.VMEM((NG, D), jnp.float32),
      ],
  )(a0, a1, x, wrT, br_, wrootT, g_, b_, batch3)


def _tail_body(sums_ref, cnt_ref, mw_ref, frac_ref, smol_ref,
               sx_ref, ssrc_ref, sdst_ref, sew_ref,
               fcmT_ref, fcl_ref, fcb_ref,
               swr1T_ref, sbr1_ref, swroot1T_ref,
               swr2T_ref, sbr2_ref, swroot2T_ref,
               sg_ref, sb_ref, sfcT_ref, sfcb_ref,
               qT_ref, qb_ref, kT_ref, kb_ref, vT_ref, vb_ref,
               r0aT_ref, r0bT_ref, r0c_ref, r0b_ref,
               r1T_ref, r1b_ref, r2T_ref, r2b_ref, o_ref):
  cnt = jnp.maximum(cnt_ref[...], 1.0)
  xmean = sums_ref[...] / cnt
  logmw = jnp.log(mw_ref[...]) * (1.0 / jnp.log(10.0))
  emb = jnp.tanh(jnp.dot(xmean, fcmT_ref[...], preferred_element_type=jnp.float32)
                 + jnp.dot(logmw, fcl_ref[...], preferred_element_type=jnp.float32)
                 + fcb_ref[...])

  sx = sx_ref[...]
  ssrc = ssrc_ref[0, :]
  sdst = sdst_ref[0, :]
  sew = sew_ref[...]
  src_oh = (ssrc[:, None] == lax.broadcasted_iota(
      jnp.int32, (SE_PAD, SN_PAD), 1)).astype(jnp.float32)
  dst_oh = (sdst[:, None] == lax.broadcasted_iota(
      jnp.int32, (SE_PAD, SN_PAD), 1)).astype(jnp.float32)

  def sconv(xin, wrT, brr, wrootT):
    msg = jnp.dot(src_oh, xin, preferred_element_type=jnp.float32) * sew
    aggr = lax.dot_general(dst_oh, msg, (((0,), (0,)), ((), ())),
                           preferred_element_type=jnp.float32)
    y = jnp.dot(aggr, wrT, preferred_element_type=jnp.float32) + brr
    y = y + jnp.dot(xin, wrootT, preferred_element_type=jnp.float32)
    mu = jnp.mean(y, axis=1, keepdims=True)
    var = jnp.mean((y - mu) ** 2, axis=1, keepdims=True)
    y = (y - mu) * lax.rsqrt(var + 1e-5) * sg_ref[...] + sb_ref[...]
    return jnp.maximum(y, 0.0)

  h = sconv(sx, swr1T_ref[...], sbr1_ref[...], swroot1T_ref[...])
  h = sconv(h, swr2T_ref[...], sbr2_ref[...], swroot2T_ref[...])
  rmask = (lax.broadcasted_iota(jnp.int32, (SN_PAD, 1), 0) < SN)
  h = jnp.where(rmask, h, 0.0)
  spool = jnp.sum(h, axis=0, keepdims=True) * (1.0 / SN)
  semb = jnp.tanh(jnp.dot(spool, sfcT_ref[...],
                          preferred_element_type=jnp.float32) + sfcb_ref[...])

  q = jnp.dot(emb, qT_ref[...], preferred_element_type=jnp.float32) + qb_ref[...]
  kk = jnp.dot(emb, kT_ref[...], preferred_element_type=jnp.float32) + kb_ref[...]
  v = jnp.dot(emb, vT_ref[...], preferred_element_type=jnp.float32) + vb_ref[...]
  scores = lax.dot_general(q, kk, (((1,), (1,)), ((), ())),
                           preferred_element_type=jnp.float32) * 0.125
  m = jnp.max(scores, axis=0, keepdims=True)
  e = jnp.exp(scores - m)
  att = e / jnp.sum(e, axis=0, keepdims=True)
  att_out = jnp.dot(att, v, preferred_element_type=jnp.float32)
  xx = jnp.dot(frac_ref[...], att_out, preferred_element_type=jnp.float32)

  h0 = jnp.dot(xx, r0aT_ref[...], preferred_element_type=jnp.float32)
  h0 = h0 + jnp.dot(semb, r0bT_ref[...], preferred_element_type=jnp.float32)
  h0 = h0 + smol_ref[...] * r0c_ref[...] + r0b_ref[...]
  h0 = jnp.maximum(h0, 0.0)
  h1 = jnp.maximum(jnp.dot(h0, r1T_ref[...],
                           preferred_element_type=jnp.float32) + r1b_ref[...], 0.0)
  o_ref[...] = jnp.dot(h1, r2T_ref[...],
                       preferred_element_type=jnp.float32) + r2b_ref[...]


def _tail(*args):
  return pl.pallas_call(
      _tail_body,
      out_shape=jax.ShapeDtypeStruct((1, 1), jnp.float32),
  )(*args)


def kernel(x, edge_index, edge_attr, batch, mw, frac, salt_mol, salt_x,
           salt_edge_index, salt_edge_attr, salt_batch, params):
  del salt_batch
  src = edge_index[0]
  dst = edge_index[1]
  ew = edge_attr

  p = params["phi"]
  (wr1, br1, wroot1), (wr2, br2, wroot2) = p["convs"]
  g_ = p["ln_g"][None, :]
  b_ = p["ln_b"][None, :]

  aggr1 = _edge_aggregate(x, src, dst, ew)
  h1 = _conv_dense(aggr1, x, wr1.T, br1[None, :], wroot1.T, g_, b_)
  aggr2 = _edge_aggregate(h1, src, dst, ew)
  batch3 = batch.reshape(GRID, 1, BR)
  sums, cnt = _conv_pool(aggr2, h1, wr2.T, br2[None, :], wroot2.T, g_, b_,
                         batch3)

  sx = jnp.zeros((SN_PAD, D), jnp.float32).at[:SN].set(salt_x)
  ssrc = jnp.full((SE_PAD,), SN_PAD - 1, jnp.int32).at[:SE].set(
      salt_edge_index[0])[None, :]
  sdst = jnp.full((SE_PAD,), SN_PAD - 1, jnp.int32).at[:SE].set(
      salt_edge_index[1])[None, :]
  sew = jnp.zeros((SE_PAD,), jnp.float32).at[:SE].set(salt_edge_attr)[:, None]

  ps = params["phi_salt"]
  (swr1, sbr1, swroot1), (swr2, sbr2, swroot2) = ps["convs"]

  fc_W = p["fc_W"]
  rho0_W = params["rho0_W"]

  out = _tail(
      sums, cnt, mw[:, None], frac[None, :], jnp.reshape(salt_mol, (1, 1)),
      sx, ssrc, sdst, sew,
      fc_W[:, :D].T, fc_W[:, D][None, :], p["fc_b"][None, :],
      swr1.T, sbr1[None, :], swroot1.T,
      swr2.T, sbr2[None, :], swroot2.T,
      ps["ln_g"][None, :], ps["ln_b"][None, :],
      ps["fc_W"].T, ps["fc_b"][None, :],
      params["att_q_W"].T, params["att_q_b"][None, :],
      params["att_k_W"].T, params["att_k_b"][None, :],
      params["att_v_W"].T, params["att_v_b"][None, :],
      rho0_W[:, :64].T, rho0_W[:, 64:128].T, rho0_W[:, 128][None, :],
      params["rho0_b"][None, :],
      params["rho1_W"].T, params["rho1_b"][None, :],
      params["rho2_W"].T, params["rho2_b"][None, :],
  )
  return out.reshape(1)


# speedup vs baseline: 8.0491x; 2.0256x over previous
import functools

import jax
import jax.numpy as jnp
from jax import lax
from jax.experimental import pallas as pl
from jax.experimental.pallas import tpu as pltpu
from jax.experimental.pallas import tpu_sc as plsc

N_NODES = 10000
N_EDGES = 320000
D = 128
NG = 32
SN = 50
SE = 100
SN_PAD = 64
SE_PAD = 128

NW = 32
EPW = N_EDGES // NW
CHUNK = 80
NCHUNK = EPW // CHUNK


def _edge_aggregate(x, src3, dst3, ew3):
  mesh = plsc.VectorSubcoreMesh(core_axis_name="c", subcore_axis_name="s")

  @functools.partial(
      pl.kernel,
      mesh=mesh,
      out_type=jax.ShapeDtypeStruct((2 * N_NODES, D), jnp.float32),
      scratch_types=[
          pltpu.VMEM((2, CHUNK), jnp.int32),
          pltpu.VMEM((NCHUNK, CHUNK), jnp.int32),
          pltpu.VMEM((2, CHUNK), jnp.float32),
          pltpu.VMEM((CHUNK, D), jnp.float32),
          pltpu.VMEM((CHUNK, D), jnp.float32),
          pltpu.VMEM_SHARED((N_NODES, D), jnp.float32),
          pltpu.SemaphoreType.DMA,
          pltpu.SemaphoreType.DMA,
          pltpu.SemaphoreType.DMA,
          pltpu.SemaphoreType.DMA,
          pltpu.SemaphoreType.DMA,
          pltpu.SemaphoreType.DMA,
      ],
  )
  def k(x_hbm, src_hbm, dst_hbm, ew_hbm, out_hbm,
        srcbuf_v, dstall_v, ewbuf_v, rows0_v, rows1_v,
        aggr_sh, gsem0, gsem1, isem0, isem1, ssem0, ssem1):
    cid = lax.axis_index("c")
    sid = lax.axis_index("s")
    wid = sid * 2 + cid

    zeros = jnp.zeros((16,), jnp.float32)

    def zrow(r, _):
      for j in range(8):
        rows0_v[r, pl.ds(j * 16, 16)] = zeros
      return 0
    lax.fori_loop(0, CHUNK, zrow, 0)

    def zinit(i, _):
      c = i * 16 + sid

      @pl.when(c < NCHUNK)
      def _():
        pltpu.sync_copy(rows0_v, aggr_sh.at[pl.ds(c * CHUNK, CHUNK)])
      return 0
    lax.fori_loop(0, 8, zinit, 0)

    base = wid * NCHUNK
    pltpu.sync_copy(dst_hbm.at[wid], dstall_v)
    plsc.subcore_barrier()

    rows = (rows0_v, rows1_v)
    gsems = (gsem0, gsem1)
    isems = (isem0, isem1)
    ssems = (ssem0, ssem1)

    def issue_idx(c, k):
      pltpu.async_copy(src_hbm.at[base + c], srcbuf_v.at[pl.ds(k, 1)],
                       isems[k])
      pltpu.async_copy(ew_hbm.at[base + c], ewbuf_v.at[pl.ds(k, 1)], isems[k])

    def wait_idx(c, k):
      pltpu.make_async_copy(src_hbm.at[base + c], srcbuf_v.at[pl.ds(k, 1)],
                            isems[k]).wait()
      pltpu.make_async_copy(ew_hbm.at[base + c], ewbuf_v.at[pl.ds(k, 1)],
                            isems[k]).wait()

    def issue_gather(c, k):
      pltpu.async_copy(x_hbm.at[srcbuf_v.at[k]], rows[k], gsems[k])

    def wait_gather(c, k):
      pltpu.make_async_copy(x_hbm.at[srcbuf_v.at[k]], rows[k],
                            gsems[k]).wait()

    def scale(c, k):
      rv = rows[k]

      def scale_group(g, _):
        wv = ewbuf_v[k, pl.ds(g * 16, 16)]
        for u in range(16):
          e = g * 16 + u
          w = lax.gather(
              wv, jnp.full((16, 1), u, jnp.int32),
              lax.GatherDimensionNumbers(offset_dims=(),
                                         collapsed_slice_dims=(0,),
                                         start_index_map=(0,)),
              slice_sizes=(1,),
              mode=lax.GatherScatterMode.PROMISE_IN_BOUNDS)
          for j in range(8):
            rv[e, pl.ds(j * 16, 16)] = rv[e, pl.ds(j * 16, 16)] * w
        return 0
      lax.fori_loop(0, CHUNK // 16, scale_group, 0)

    def scatter(c, k):
      pltpu.sync_copy(rows[k], aggr_sh.at[dstall_v.at[c]], add=True)

    issue_idx(0, 0)
    wait_idx(0, 0)
    issue_gather(0, 0)
    issue_idx(1, 1)

    def pipe_body(i, _):
      cA = 2 * i
      wait_gather(cA, 0)

      @pl.when(cA + 1 < NCHUNK)
      def _():
        wait_idx(cA + 1, 1)
        issue_gather(cA + 1, 1)
      scale(cA, 0)
      scatter(cA, 0)

      @pl.when(cA + 2 < NCHUNK)
      def _():
        issue_idx(cA + 2, 0)

      @pl.when(cA + 1 < NCHUNK)
      def _():
        wait_gather(cA + 1, 1)

        @pl.when(cA + 2 < NCHUNK)
        def _():
          wait_idx(cA + 2, 0)
          issue_gather(cA + 2, 0)
        scale(cA + 1, 1)
        scatter(cA + 1, 1)

        @pl.when(cA + 3 < NCHUNK)
        def _():
          issue_idx(cA + 3, 1)
      return 0
    lax.fori_loop(0, (NCHUNK + 1) // 2, pipe_body, 0)

    plsc.subcore_barrier()
    obase = cid * N_NODES
    for kk in range(3):
      pltpu.sync_copy(aggr_sh.at[pl.ds(kk * 3200 + sid * 200, 200)],
                      out_hbm.at[pl.ds(obase + kk * 3200 + sid * 200, 200)])

    @pl.when(sid < 2)
    def _():
      pltpu.sync_copy(aggr_sh.at[pl.ds(9600 + sid * 200, 200)],
                      out_hbm.at[pl.ds(obase + 9600 + sid * 200, 200)])

  return k(x, src3, dst3, ew3)


BR = 2000
GRID = N_NODES // BR


def _conv_body(a0_ref, a1_ref, x_ref, wr_ref, br_ref, wroot_ref, g_ref, b_ref,
               o_ref):
  a = a0_ref[...] + a1_ref[...]
  y = jnp.dot(a, wr_ref[...], preferred_element_type=jnp.float32)
  y = y + jnp.dot(x_ref[...], wroot_ref[...], preferred_element_type=jnp.float32)
  y = y + br_ref[...]
  mu = jnp.mean(y, axis=1, keepdims=True)
  var = jnp.mean((y - mu) ** 2, axis=1, keepdims=True)
  y = (y - mu) * lax.rsqrt(var + 1e-5) * g_ref[...] + b_ref[...]
  o_ref[...] = jnp.maximum(y, 0.0)


def _conv_dense(aggr2, x, wrT, br_, wrootT, g_, b_):
  a0 = aggr2[:N_NODES]
  a1 = aggr2[N_NODES:]
  return pl.pallas_call(
      _conv_body,
      grid=(GRID,),
      in_specs=[
          pl.BlockSpec((BR, D), lambda i: (i, 0)),
          pl.BlockSpec((BR, D), lambda i: (i, 0)),
          pl.BlockSpec((BR, D), lambda i: (i, 0)),
          pl.BlockSpec((D, D), lambda i: (0, 0)),
          pl.BlockSpec((1, D), lambda i: (0, 0)),
          pl.BlockSpec((D, D), lambda i: (0, 0)),
          pl.BlockSpec((1, D), lambda i: (0, 0)),
          pl.BlockSpec((1, D), lambda i: (0, 0)),
      ],
      out_specs=pl.BlockSpec((BR, D), lambda i: (i, 0)),
      out_shape=jax.ShapeDtypeStruct((N_NODES, D), jnp.float32),
  )(a0, a1, x, wrT, br_, wrootT, g_, b_)


def _conv_pool_body(a0_ref, a1_ref, x_ref, wr_ref, br_ref, wroot_ref, g_ref,
                    b_ref, bat_ref, sums_ref, cnt_ref, acc, cacc):
  i = pl.program_id(0)
  a = a0_ref[...] + a1_ref[...]
  y = jnp.dot(a, wr_ref[...], preferred_element_type=jnp.float32)
  y = y + jnp.dot(x_ref[...], wroot_ref[...], preferred_element_type=jnp.float32)
  y = y + br_ref[...]
  mu = jnp.mean(y, axis=1, keepdims=True)
  var = jnp.mean((y - mu) ** 2, axis=1, keepdims=True)
  y = (y - mu) * lax.rsqrt(var + 1e-5) * g_ref[...] + b_ref[...]
  h = jnp.maximum(y, 0.0)

  bat = bat_ref[0, 0, :]
  onehot = (bat[:, None] == lax.broadcasted_iota(jnp.int32, (BR, NG), 1))
  onehot = onehot.astype(jnp.float32)
  psum = lax.dot_general(onehot, h, (((0,), (0,)), ((), ())),
                         preferred_element_type=jnp.float32)
  pcnt = lax.dot_general(onehot, jnp.ones((BR, D), jnp.float32),
                         (((0,), (0,)), ((), ())),
                         preferred_element_type=jnp.float32)

  @pl.when(i == 0)
  def _():
    acc[...] = jnp.zeros_like(acc)
    cacc[...] = jnp.zeros_like(cacc)

  acc[...] += psum
  cacc[...] += pcnt

  @pl.when(i == GRID - 1)
  def _():
    sums_ref[...] = acc[...]
    cnt_ref[...] = cacc[...]


def _conv_pool(aggr2, x, wrT, br_, wrootT, g_, b_, batch3):
  a0 = aggr2[:N_NODES]
  a1 = aggr2[N_NODES:]
  return pl.pallas_call(
      _conv_pool_body,
      grid=(GRID,),
      in_specs=[
          pl.BlockSpec((BR, D), lambda i: (i, 0)),
          pl.BlockSpec((BR, D), lambda i: (i, 0)),
          pl.BlockSpec((BR, D), lambda i: (i, 0)),
          pl.BlockSpec((D, D), lambda i: (0, 0)),
          pl.BlockSpec((1, D), lambda i: (0, 0)),
          pl.BlockSpec((D, D), lambda i: (0, 0)),
          pl.BlockSpec((1, D), lambda i: (0, 0)),
          pl.BlockSpec((1, D), lambda i: (0, 0)),
          pl.BlockSpec((1, 1, BR), lambda i: (i, 0, 0)),
      ],
      out_specs=[
          pl.BlockSpec((NG, D), lambda i: (0, 0)),
          pl.BlockSpec((NG, D), lambda i: (0, 0)),
      ],
      out_shape=[
          jax.ShapeDtypeStruct((NG, D), jnp.float32),
          jax.ShapeDtypeStruct((NG, D), jnp.float32),
      ],
      scratch_shapes=[
          pltpu.VMEM((NG, D), jnp.float32),
          pltpu.VMEM((NG, D), jnp.float32),
      ],
  )(a0, a1, x, wrT, br_, wrootT, g_, b_, batch3)


def _tail_body(sums_ref, cnt_ref, mw_ref, frac_ref, smol_ref,
               sx_ref, ssrc_ref, sdst_ref, sew_ref,
               fcmT_ref, fcl_ref, fcb_ref,
               swr1T_ref, sbr1_ref, swroot1T_ref,
               swr2T_ref, sbr2_ref, swroot2T_ref,
               sg_ref, sb_ref, sfcT_ref, sfcb_ref,
               qT_ref, qb_ref, kT_ref, kb_ref, vT_ref, vb_ref,
               r0aT_ref, r0bT_ref, r0c_ref, r0b_ref,
               r1T_ref, r1b_ref, r2T_ref, r2b_ref, o_ref):
  cnt = jnp.maximum(cnt_ref[...], 1.0)
  xmean = sums_ref[...] / cnt
  logmw = jnp.log(mw_ref[...]) * (1.0 / jnp.log(10.0))
  emb = jnp.tanh(jnp.dot(xmean, fcmT_ref[...], preferred_element_type=jnp.float32)
                 + jnp.dot(logmw, fcl_ref[...], preferred_element_type=jnp.float32)
                 + fcb_ref[...])

  sx = sx_ref[...]
  ssrc = ssrc_ref[0, :]
  sdst = sdst_ref[0, :]
  sew = sew_ref[...]
  src_oh = (ssrc[:, None] == lax.broadcasted_iota(
      jnp.int32, (SE_PAD, SN_PAD), 1)).astype(jnp.float32)
  dst_oh = (sdst[:, None] == lax.broadcasted_iota(
      jnp.int32, (SE_PAD, SN_PAD), 1)).astype(jnp.float32)

  def sconv(xin, wrT, brr, wrootT):
    msg = jnp.dot(src_oh, xin, preferred_element_type=jnp.float32) * sew
    aggr = lax.dot_general(dst_oh, msg, (((0,), (0,)), ((), ())),
                           preferred_element_type=jnp.float32)
    y = jnp.dot(aggr, wrT, preferred_element_type=jnp.float32) + brr
    y = y + jnp.dot(xin, wrootT, preferred_element_type=jnp.float32)
    mu = jnp.mean(y, axis=1, keepdims=True)
    var = jnp.mean((y - mu) ** 2, axis=1, keepdims=True)
    y = (y - mu) * lax.rsqrt(var + 1e-5) * sg_ref[...] + sb_ref[...]
    return jnp.maximum(y, 0.0)

  h = sconv(sx, swr1T_ref[...], sbr1_ref[...], swroot1T_ref[...])
  h = sconv(h, swr2T_ref[...], sbr2_ref[...], swroot2T_ref[...])
  rmask = (lax.broadcasted_iota(jnp.int32, (SN_PAD, 1), 0) < SN)
  h = jnp.where(rmask, h, 0.0)
  spool = jnp.sum(h, axis=0, keepdims=True) * (1.0 / SN)
  semb = jnp.tanh(jnp.dot(spool, sfcT_ref[...],
                          preferred_element_type=jnp.float32) + sfcb_ref[...])

  q = jnp.dot(emb, qT_ref[...], preferred_element_type=jnp.float32) + qb_ref[...]
  kk = jnp.dot(emb, kT_ref[...], preferred_element_type=jnp.float32) + kb_ref[...]
  v = jnp.dot(emb, vT_ref[...], preferred_element_type=jnp.float32) + vb_ref[...]
  scores = lax.dot_general(q, kk, (((1,), (1,)), ((), ())),
                           preferred_element_type=jnp.float32) * 0.125
  m = jnp.max(scores, axis=0, keepdims=True)
  e = jnp.exp(scores - m)
  att = e / jnp.sum(e, axis=0, keepdims=True)
  att_out = jnp.dot(att, v, preferred_element_type=jnp.float32)
  xx = jnp.dot(frac_ref[...], att_out, preferred_element_type=jnp.float32)

  h0 = jnp.dot(xx, r0aT_ref[...], preferred_element_type=jnp.float32)
  h0 = h0 + jnp.dot(semb, r0bT_ref[...], preferred_element_type=jnp.float32)
  h0 = h0 + smol_ref[...] * r0c_ref[...] + r0b_ref[...]
  h0 = jnp.maximum(h0, 0.0)
  h1 = jnp.maximum(jnp.dot(h0, r1T_ref[...],
                           preferred_element_type=jnp.float32) + r1b_ref[...], 0.0)
  o_ref[...] = jnp.dot(h1, r2T_ref[...],
                       preferred_element_type=jnp.float32) + r2b_ref[...]


def _tail(*args):
  return pl.pallas_call(
      _tail_body,
      out_shape=jax.ShapeDtypeStruct((1, 1), jnp.float32),
  )(*args)


def kernel(x, edge_index, edge_attr, batch, mw, frac, salt_mol, salt_x,
           salt_edge_index, salt_edge_attr, salt_batch, params):
  del salt_batch
  src3 = edge_index[0].reshape(NW * NCHUNK, 1, CHUNK)
  dst3 = edge_index[1].reshape(NW, NCHUNK, CHUNK)
  ew3 = edge_attr.reshape(NW * NCHUNK, 1, CHUNK)

  p = params["phi"]
  (wr1, br1, wroot1), (wr2, br2, wroot2) = p["convs"]
  g_ = p["ln_g"][None, :]
  b_ = p["ln_b"][None, :]

  aggr1 = _edge_aggregate(x, src3, dst3, ew3)
  h1 = _conv_dense(aggr1, x, wr1.T, br1[None, :], wroot1.T, g_, b_)
  aggr2 = _edge_aggregate(h1, src3, dst3, ew3)
  batch3 = batch.reshape(GRID, 1, BR)
  sums, cnt = _conv_pool(aggr2, h1, wr2.T, br2[None, :], wroot2.T, g_, b_,
                         batch3)

  sx = jnp.zeros((SN_PAD, D), jnp.float32).at[:SN].set(salt_x)
  ssrc = jnp.full((SE_PAD,), SN_PAD - 1, jnp.int32).at[:SE].set(
      salt_edge_index[0])[None, :]
  sdst = jnp.full((SE_PAD,), SN_PAD - 1, jnp.int32).at[:SE].set(
      salt_edge_index[1])[None, :]
  sew = jnp.zeros((SE_PAD,), jnp.float32).at[:SE].set(salt_edge_attr)[:, None]

  ps = params["phi_salt"]
  (swr1, sbr1, swroot1), (swr2, sbr2, swroot2) = ps["convs"]

  fc_W = p["fc_W"]
  rho0_W = params["rho0_W"]

  out = _tail(
      sums, cnt, mw[:, None], frac[None, :], jnp.reshape(salt_mol, (1, 1)),
      sx, ssrc, sdst, sew,
      fc_W[:, :D].T, fc_W[:, D][None, :], p["fc_b"][None, :],
      swr1.T, sbr1[None, :], swroot1.T,
      swr2.T, sbr2[None, :], swroot2.T,
      ps["ln_g"][None, :], ps["ln_b"][None, :],
      ps["fc_W"].T, ps["fc_b"][None, :],
      params["att_q_W"].T, params["att_q_b"][None, :],
      params["att_k_W"].T, params["att_k_b"][None, :],
      params["att_v_W"].T, params["att_v_b"][None, :],
      rho0_W[:, :64].T, rho0_W[:, 64:128].T, rho0_W[:, 128][None, :],
      params["rho0_b"][None, :],
      params["rho1_W"].T, params["rho1_b"][None, :],
      params["rho2_W"].T, params["rho2_b"][None, :],
  )
  return out.reshape(1)
